# baseline (speedup 1.0000x reference)
import jax
import jax.numpy as jnp
from jax import lax
from jax.experimental import pallas as pl
from jax.experimental.pallas import tpu as pltpu
from jax.experimental.pallas import tpu_sc as plsc

B, H, W, C = 4, 32, 32, 384
NH, NP = 12, 4
HD = C // NH
FF = 2048
N = H * W
NJ = NP * 4
NL = NH * NJ
M_TOT = B * N * NL

NWORK = 32
M_W = M_TOT // NWORK
CH = 1024
KSUB = CH // 128
NCHUNK = M_W // CH


def _prep_body(q_ref, Wv_ref, bv_ref, Wso_ref, bso_ref, Waw_ref, baw_ref,
               val_ref, idx_ref, wgt_ref):
    b = pl.program_id(0)
    q = q_ref[0]
    value = jnp.dot(q, Wv_ref[...], preferred_element_type=jnp.float32) + bv_ref[...]
    val_ref[0] = value.astype(jnp.bfloat16)
    off = jnp.dot(q, Wso_ref[...], preferred_element_type=jnp.float32) + bso_ref[...]
    awl = jnp.dot(q, Waw_ref[...], preferred_element_type=jnp.float32) + baw_ref[...]

    m = jnp.max(awl, axis=-1, keepdims=True)
    e = jnp.exp(awl - m)
    i48r = lax.broadcasted_iota(jnp.int32, (NH * NP, NH * NP), 0)
    i48c = lax.broadcasted_iota(jnp.int32, (NH * NP, NH * NP), 1)
    gsum = (i48r // NP == i48c // NP).astype(jnp.float32)
    denom = jnp.dot(e, gsum, preferred_element_type=jnp.float32)
    awn = e / denom

    n_row = lax.broadcasted_iota(jnp.int32, (N, 1), 0)
    colf = (n_row % W).astype(jnp.float32)
    rowf = (n_row // W).astype(jnp.float32)
    l96 = lax.broadcasted_iota(jnp.int32, (1, 2 * NH * NP), 1)
    is_x = (l96 % 2) == 0
    pix = off + jnp.where(is_x, colf, rowf)
    f0 = jnp.floor(pix)
    frac = pix - f0

    p96r = lax.broadcasted_iota(jnp.int32, (2 * NH * NP, NL), 0)
    p96c = lax.broadcasted_iota(jnp.int32, (2 * NH * NP, NL), 1)
    src = (p96c // NJ) * 8 + ((p96c % NJ) // 4) * 2
    Px = (p96r == src).astype(jnp.float32)
    Py = (p96r == src + 1).astype(jnp.float32)
    x0 = jnp.dot(f0, Px, preferred_element_type=jnp.float32)
    y0 = jnp.dot(f0, Py, preferred_element_type=jnp.float32)
    fx = jnp.dot(frac, Px, preferred_element_type=jnp.float32)
    fy = jnp.dot(frac, Py, preferred_element_type=jnp.float32)

    a48r = lax.broadcasted_iota(jnp.int32, (NH * NP, NL), 0)
    a48c = lax.broadcasted_iota(jnp.int32, (NH * NP, NL), 1)
    Paw = (a48r == a48c // 4).astype(jnp.float32)
    awe = jnp.dot(awn, Paw, preferred_element_type=jnp.float32)

    l192 = lax.broadcasted_iota(jnp.int32, (1, NL), 1)
    dxv = ((l192 % 4) % 2).astype(jnp.float32)
    dyv = ((l192 % 4) // 2).astype(jnp.float32)
    hl = l192 // NJ
    xi = x0 + dxv
    yi = y0 + dyv
    valid = ((xi >= 0.0) & (xi < float(W)) & (yi >= 0.0) & (yi < float(H)))
    xc = jnp.clip(xi, 0.0, float(W - 1)).astype(jnp.int32)
    yc = jnp.clip(yi, 0.0, float(H - 1)).astype(jnp.int32)
    idx_ref[0] = ((b * H + yc) * W + xc) * NH + hl
    wx = jnp.where(dxv == 0.0, 1.0 - fx, fx)
    wy = jnp.where(dyv == 0.0, 1.0 - fy, fy)
    wgt_ref[0] = awe * wx * wy * jnp.where(valid, 1.0, 0.0)


def _sc_gather_body(table_ref, idx_ref, g_ref, idx_v, rows_v, sem):
    wid = lax.axis_index("s") * 2 + lax.axis_index("c")
    base = wid * M_W

    def chunk(i, carry):
        cbase = base + i * CH
        pltpu.sync_copy(idx_ref.at[pl.ds(pl.multiple_of(cbase // 128, 8), KSUB)],
                        idx_v)
        copies = [
            pltpu.make_async_copy(table_ref.at[idx_v.at[k]],
                                  rows_v.at[pl.ds(k * 128, 128)], sem)
            for k in range(KSUB)
        ]
        for cp in copies:
            cp.start()
        for cp in copies:
            cp.wait()
        pltpu.sync_copy(rows_v, g_ref.at[pl.ds(cbase, CH)])
        return carry

    lax.fori_loop(0, NCHUNK, chunk, 0)


def _ln(x, g, b):
    m = jnp.mean(x, axis=-1, keepdims=True)
    xc = x - m
    v = jnp.mean(xc * xc, axis=-1, keepdims=True)
    return xc * lax.rsqrt(v + 1e-5) * g + b


def _reduce_ffn_body(g_ref, w_ref, q_ref, Wo_ref, bo_ref, W1_ref, b1_ref,
                     W2_ref, b2_ref, g1_ref, be1_ref, g2_ref, be2_ref, out_ref):
    q = q_ref[0]
    g = g_ref[0]
    w = w_ref[0]
    er = lax.broadcasted_iota(jnp.int32, (NJ, NJ * HD), 0)
    ec = lax.broadcasted_iota(jnp.int32, (NJ, NJ * HD), 1)
    E16 = (er == ec // HD).astype(jnp.float32)
    sr = lax.broadcasted_iota(jnp.int32, (NJ * HD, HD), 0)
    sc = lax.broadcasted_iota(jnp.int32, (NJ * HD, HD), 1)
    S512 = (sr % HD == sc).astype(jnp.float32)
    parts = []
    for h in range(NH):
        wh = w[:, h * NJ:(h + 1) * NJ]
        gh = g[:, h * NJ * HD:(h + 1) * NJ * HD]
        wE = jnp.dot(wh, E16, preferred_element_type=jnp.float32)
        parts.append(jnp.dot(wE * gh.astype(jnp.float32), S512,
                             preferred_element_type=jnp.float32))
    attn = jnp.concatenate(parts, axis=1)
    src2 = jnp.dot(attn, Wo_ref[...], preferred_element_type=jnp.float32) + bo_ref[...]
    h1 = _ln(q + src2, g1_ref[...], be1_ref[...])
    f = jnp.maximum(jnp.dot(h1, W1_ref[...], preferred_element_type=jnp.float32)
                    + b1_ref[...], 0.0)
    ff = jnp.dot(f, W2_ref[...], preferred_element_type=jnp.float32) + b2_ref[...]
    out_ref[0] = _ln(h1 + ff, g2_ref[...], be2_ref[...])


def kernel(src, Wso, bso, Waw, baw, Wv, bv, Wo, bo, W1, b1, W2, b2, g1, be1, g2, be2):
    q3 = src.reshape(B, N, C)

    full = lambda shape: pl.BlockSpec(shape, lambda *a: (0,) * len(shape))
    value, idx, wgt = pl.pallas_call(
        _prep_body,
        grid=(B,),
        in_specs=[
            pl.BlockSpec((1, N, C), lambda b: (b, 0, 0)),
            full((C, C)), full((1, C)),
            full((C, NH * NP * 2)), full((1, NH * NP * 2)),
            full((C, NH * NP)), full((1, NH * NP)),
        ],
        out_specs=[
            pl.BlockSpec((1, N, C), lambda b: (b, 0, 0)),
            pl.BlockSpec((1, N, NL), lambda b: (b, 0, 0)),
            pl.BlockSpec((1, N, NL), lambda b: (b, 0, 0)),
        ],
        out_shape=[
            jax.ShapeDtypeStruct((B, N, C), jnp.bfloat16),
            jax.ShapeDtypeStruct((B, N, NL), jnp.int32),
            jax.ShapeDtypeStruct((B, N, NL), jnp.float32),
        ],
    )(q3, Wv, bv.reshape(1, C), Wso, bso.reshape(1, -1), Waw, baw.reshape(1, -1))

    table = value.reshape(B * N * NH, HD)
    idx2 = idx.reshape(M_TOT // 128, 128)

    sc_gather = pl.kernel(
        _sc_gather_body,
        out_type=jax.ShapeDtypeStruct((M_TOT, HD), jnp.bfloat16),
        mesh=plsc.VectorSubcoreMesh(core_axis_name="c", subcore_axis_name="s",
                                    num_cores=2, num_subcores=16),
        scratch_types=[
            pltpu.VMEM((KSUB, 128), jnp.int32),
            pltpu.VMEM((CH, HD), jnp.bfloat16),
            pltpu.SemaphoreType.DMA,
        ],
        compiler_params=pltpu.CompilerParams(use_tc_tiling_on_sc=False),
    )
    g = sc_gather(table, idx2)

    g3 = g.reshape(B, N, NL * HD)

    RB = 256
    out = pl.pallas_call(
        _reduce_ffn_body,
        grid=(B, N // RB),
        in_specs=[
            pl.BlockSpec((1, RB, NL * HD), lambda b, i: (b, i, 0)),
            pl.BlockSpec((1, RB, NL), lambda b, i: (b, i, 0)),
            pl.BlockSpec((1, RB, C), lambda b, i: (b, i, 0)),
            full((C, C)), full((1, C)),
            full((C, FF)), full((1, FF)),
            full((FF, C)), full((1, C)),
            full((1, C)), full((1, C)), full((1, C)), full((1, C)),
        ],
        out_specs=pl.BlockSpec((1, RB, C), lambda b, i: (b, i, 0)),
        out_shape=jax.ShapeDtypeStruct((B, N, C), jnp.float32),
    )(g3, wgt, q3, Wo, bo.reshape(1, C), W1, b1.reshape(1, FF), W2,
      b2.reshape(1, C), g1.reshape(1, C), be1.reshape(1, C), g2.reshape(1, C),
      be2.reshape(1, C))
    return out


# speedup vs baseline: 1.1704x; 1.1704x over previous
import jax
import jax.numpy as jnp
from jax import lax
from jax.experimental import pallas as pl
from jax.experimental.pallas import tpu as pltpu
from jax.experimental.pallas import tpu_sc as plsc

B, H, W, C = 4, 32, 32, 384
NH, NP = 12, 4
HD = C // NH
FF = 2048
N = H * W
NJ = NP * 4
NL = NH * NJ
M_TOT = B * N * NL

NWORK = 32
M_B = M_TOT // B
M_W = M_B // NWORK
CH = 1024
KSUB = CH // 128
NCHUNK = M_W // CH


def _prep_body(q_ref, Wv_ref, bv_ref, Wso_ref, bso_ref, Waw_ref, baw_ref,
               val_ref, idx_ref, wgt_ref):
    b = pl.program_id(0)
    q = q_ref[0]
    value = jnp.dot(q, Wv_ref[...], preferred_element_type=jnp.float32) + bv_ref[...]
    val_ref[0] = value
    off = jnp.dot(q, Wso_ref[...], preferred_element_type=jnp.float32) + bso_ref[...]
    awl = jnp.dot(q, Waw_ref[...], preferred_element_type=jnp.float32) + baw_ref[...]

    m = jnp.max(awl, axis=-1, keepdims=True)
    e = jnp.exp(awl - m)
    i48r = lax.broadcasted_iota(jnp.int32, (NH * NP, NH * NP), 0)
    i48c = lax.broadcasted_iota(jnp.int32, (NH * NP, NH * NP), 1)
    gsum = (i48r // NP == i48c // NP).astype(jnp.float32)
    denom = jnp.dot(e, gsum, preferred_element_type=jnp.float32)
    awn = e / denom

    n_row = lax.broadcasted_iota(jnp.int32, (N, 1), 0)
    colf = (n_row % W).astype(jnp.float32)
    rowf = (n_row // W).astype(jnp.float32)
    l96 = lax.broadcasted_iota(jnp.int32, (1, 2 * NH * NP), 1)
    is_x = (l96 % 2) == 0
    pix = off + jnp.where(is_x, colf, rowf)
    f0 = jnp.floor(pix)
    frac = pix - f0

    p96r = lax.broadcasted_iota(jnp.int32, (2 * NH * NP, NL), 0)
    p96c = lax.broadcasted_iota(jnp.int32, (2 * NH * NP, NL), 1)
    src = (p96c // NJ) * 8 + ((p96c % NJ) // 4) * 2
    Px = (p96r == src).astype(jnp.float32)
    Py = (p96r == src + 1).astype(jnp.float32)
    x0 = jnp.dot(f0, Px, preferred_element_type=jnp.float32)
    y0 = jnp.dot(f0, Py, preferred_element_type=jnp.float32)
    fx = jnp.dot(frac, Px, preferred_element_type=jnp.float32)
    fy = jnp.dot(frac, Py, preferred_element_type=jnp.float32)

    a48r = lax.broadcasted_iota(jnp.int32, (NH * NP, NL), 0)
    a48c = lax.broadcasted_iota(jnp.int32, (NH * NP, NL), 1)
    Paw = (a48r == a48c // 4).astype(jnp.float32)
    awe = jnp.dot(awn, Paw, preferred_element_type=jnp.float32)

    l192 = lax.broadcasted_iota(jnp.int32, (1, NL), 1)
    dxv = ((l192 % 4) % 2).astype(jnp.float32)
    dyv = ((l192 % 4) // 2).astype(jnp.float32)
    hl = l192 // NJ
    xi = x0 + dxv
    yi = y0 + dyv
    valid = ((xi >= 0.0) & (xi < float(W)) & (yi >= 0.0) & (yi < float(H)))
    xc = jnp.clip(xi, 0.0, float(W - 1)).astype(jnp.int32)
    yc = jnp.clip(yi, 0.0, float(H - 1)).astype(jnp.int32)
    idx_ref[0] = ((b * H + yc) * W + xc) * NH + hl
    wx = jnp.where(dxv == 0.0, 1.0 - fx, fx)
    wy = jnp.where(dyv == 0.0, 1.0 - fy, fy)
    wgt_ref[0] = awe * wx * wy * jnp.where(valid, 1.0, 0.0)


def _sc_gather_body(table_ref, idx_ref, g_ref, idx_v, rows_v, sem):
    wid = lax.axis_index("s") * 2 + lax.axis_index("c")
    base = wid * M_W

    def chunk(i, carry):
        cbase = base + i * CH
        pltpu.sync_copy(idx_ref.at[pl.ds(pl.multiple_of(cbase // 128, 8), KSUB)],
                        idx_v)
        copies = [
            pltpu.make_async_copy(table_ref.at[idx_v.at[k]],
                                  rows_v.at[pl.ds(k * 128, 128)], sem)
            for k in range(KSUB)
        ]
        for cp in copies:
            cp.start()
        for cp in copies:
            cp.wait()
        pltpu.sync_copy(rows_v, g_ref.at[pl.ds(cbase, CH)])
        return carry

    lax.fori_loop(0, NCHUNK, chunk, 0)


def _ln(x, g, b):
    m = jnp.mean(x, axis=-1, keepdims=True)
    xc = x - m
    v = jnp.mean(xc * xc, axis=-1, keepdims=True)
    return xc * lax.rsqrt(v + 1e-5) * g + b


def _reduce_ffn_body(g_ref, w_ref, q_ref, Wo_ref, bo_ref, W1_ref, b1_ref,
                     W2_ref, b2_ref, g1_ref, be1_ref, g2_ref, be2_ref, out_ref):
    q = q_ref[0]
    g = g_ref[0]
    w = w_ref[0]
    er = lax.broadcasted_iota(jnp.int32, (NJ, NJ * HD), 0)
    ec = lax.broadcasted_iota(jnp.int32, (NJ, NJ * HD), 1)
    E16 = (er == ec // HD).astype(jnp.float32)
    sr = lax.broadcasted_iota(jnp.int32, (NJ * HD, HD), 0)
    sc = lax.broadcasted_iota(jnp.int32, (NJ * HD, HD), 1)
    S512 = (sr % HD == sc).astype(jnp.float32)
    parts = []
    for h in range(NH):
        wh = w[:, h * NJ:(h + 1) * NJ]
        gh = g[:, h * NJ * HD:(h + 1) * NJ * HD]
        wE = jnp.dot(wh, E16, preferred_element_type=jnp.float32)
        parts.append(jnp.dot(wE * gh, S512,
                             preferred_element_type=jnp.float32))
    attn = jnp.concatenate(parts, axis=1)
    src2 = jnp.dot(attn, Wo_ref[...], preferred_element_type=jnp.float32) + bo_ref[...]
    h1 = _ln(q + src2, g1_ref[...], be1_ref[...])
    f = jnp.maximum(jnp.dot(h1, W1_ref[...], preferred_element_type=jnp.float32)
                    + b1_ref[...], 0.0)
    ff = jnp.dot(f, W2_ref[...], preferred_element_type=jnp.float32) + b2_ref[...]
    out_ref[0] = _ln(h1 + ff, g2_ref[...], be2_ref[...])


def kernel(src, Wso, bso, Waw, baw, Wv, bv, Wo, bo, W1, b1, W2, b2, g1, be1, g2, be2):
    q3 = src.reshape(B, N, C)

    full = lambda shape: pl.BlockSpec(shape, lambda *a: (0,) * len(shape))
    value, idx, wgt = pl.pallas_call(
        _prep_body,
        grid=(B,),
        in_specs=[
            pl.BlockSpec((1, N, C), lambda b: (b, 0, 0)),
            full((C, C)), full((1, C)),
            full((C, NH * NP * 2)), full((1, NH * NP * 2)),
            full((C, NH * NP)), full((1, NH * NP)),
        ],
        out_specs=[
            pl.BlockSpec((1, N, C), lambda b: (b, 0, 0)),
            pl.BlockSpec((1, N, NL), lambda b: (b, 0, 0)),
            pl.BlockSpec((1, N, NL), lambda b: (b, 0, 0)),
        ],
        out_shape=[
            jax.ShapeDtypeStruct((B, N, C), jnp.float32),
            jax.ShapeDtypeStruct((B, N, NL), jnp.int32),
            jax.ShapeDtypeStruct((B, N, NL), jnp.float32),
        ],
    )(q3, Wv, bv.reshape(1, C), Wso, bso.reshape(1, -1), Waw, baw.reshape(1, -1))

    table = value.reshape(B * N * NH, HD)

    sc_gather = pl.kernel(
        _sc_gather_body,
        out_type=jax.ShapeDtypeStruct((M_B, HD), jnp.float32),
        mesh=plsc.VectorSubcoreMesh(core_axis_name="c", subcore_axis_name="s",
                                    num_cores=2, num_subcores=16),
        scratch_types=[
            pltpu.VMEM((KSUB, 128), jnp.int32),
            pltpu.VMEM((CH, HD), jnp.float32),
            pltpu.SemaphoreType.DMA,
        ],
        compiler_params=pltpu.CompilerParams(use_tc_tiling_on_sc=False),
    )

    RB = 256
    outs = []
    for b in range(B):
        idx_b = lax.slice_in_dim(idx, b, b + 1, axis=0).reshape(M_B // 128, 128)
        w_b = lax.slice_in_dim(wgt, b, b + 1, axis=0)
        q_b = lax.slice_in_dim(q3, b, b + 1, axis=0)
        g3 = sc_gather(table, idx_b).reshape(1, N, NL * HD)
        outs.append(pl.pallas_call(
            _reduce_ffn_body,
            grid=(1, N // RB),
            in_specs=[
                pl.BlockSpec((1, RB, NL * HD), lambda b, i: (b, i, 0)),
                pl.BlockSpec((1, RB, NL), lambda b, i: (b, i, 0)),
                pl.BlockSpec((1, RB, C), lambda b, i: (b, i, 0)),
                full((C, C)), full((1, C)),
                full((C, FF)), full((1, FF)),
                full((FF, C)), full((1, C)),
                full((1, C)), full((1, C)), full((1, C)), full((1, C)),
            ],
            out_specs=pl.BlockSpec((1, RB, C), lambda b, i: (b, i, 0)),
            out_shape=jax.ShapeDtypeStruct((1, N, C), jnp.float32),
        )(g3, w_b, q_b, Wo, bo.reshape(1, C), W1, b1.reshape(1, FF), W2,
          b2.reshape(1, C), g1.reshape(1, C), be1.reshape(1, C),
          g2.reshape(1, C), be2.reshape(1, C)))
    return jnp.concatenate(outs, axis=0)


# speedup vs baseline: 1.3815x; 1.1804x over previous
import jax
import jax.numpy as jnp
from jax import lax
from jax.experimental import pallas as pl
from jax.experimental.pallas import tpu as pltpu
from jax.experimental.pallas import tpu_sc as plsc

B, H, W, C = 4, 32, 32, 384
NH, NP = 12, 4
HD = C // NH
FF = 2048
N = H * W
NJ = NP * 4
NL = NH * NJ
M_TOT = B * N * NL

NWORK = 32
M_W = M_TOT // NWORK
CH = 2048
KSUB = CH // 128
NCHUNK = M_W // CH


def _prep_body(q_ref, Wv_ref, bv_ref, Wso_ref, bso_ref, Waw_ref, baw_ref,
               val_ref, idx_ref, wgt_ref):
    b = pl.program_id(0)
    q = q_ref[0]
    value = jnp.dot(q, Wv_ref[...], preferred_element_type=jnp.float32) + bv_ref[...]
    val_ref[0] = value
    off = jnp.dot(q, Wso_ref[...], preferred_element_type=jnp.float32) + bso_ref[...]
    awl = jnp.dot(q, Waw_ref[...], preferred_element_type=jnp.float32) + baw_ref[...]

    m = jnp.max(awl, axis=-1, keepdims=True)
    e = jnp.exp(awl - m)
    i48r = lax.broadcasted_iota(jnp.int32, (NH * NP, NH * NP), 0)
    i48c = lax.broadcasted_iota(jnp.int32, (NH * NP, NH * NP), 1)
    gsum = (i48r // NP == i48c // NP).astype(jnp.float32)
    denom = jnp.dot(e, gsum, preferred_element_type=jnp.float32)
    awn = e / denom

    n_row = lax.broadcasted_iota(jnp.int32, (N, 1), 0)
    colf = (n_row % W).astype(jnp.float32)
    rowf = (n_row // W).astype(jnp.float32)
    l96 = lax.broadcasted_iota(jnp.int32, (1, 2 * NH * NP), 1)
    is_x = (l96 % 2) == 0
    pix = off + jnp.where(is_x, colf, rowf)
    f0 = jnp.floor(pix)
    frac = pix - f0

    p96r = lax.broadcasted_iota(jnp.int32, (2 * NH * NP, NL), 0)
    p96c = lax.broadcasted_iota(jnp.int32, (2 * NH * NP, NL), 1)
    src = (p96c // NJ) * 8 + ((p96c % NJ) // 4) * 2
    Px = (p96r == src).astype(jnp.float32)
    Py = (p96r == src + 1).astype(jnp.float32)
    x0 = jnp.dot(f0, Px, preferred_element_type=jnp.float32)
    y0 = jnp.dot(f0, Py, preferred_element_type=jnp.float32)
    fx = jnp.dot(frac, Px, preferred_element_type=jnp.float32)
    fy = jnp.dot(frac, Py, preferred_element_type=jnp.float32)

    a48r = lax.broadcasted_iota(jnp.int32, (NH * NP, NL), 0)
    a48c = lax.broadcasted_iota(jnp.int32, (NH * NP, NL), 1)
    Paw = (a48r == a48c // 4).astype(jnp.float32)
    awe = jnp.dot(awn, Paw, preferred_element_type=jnp.float32)

    l192 = lax.broadcasted_iota(jnp.int32, (1, NL), 1)
    dxv = ((l192 % 4) % 2).astype(jnp.float32)
    dyv = ((l192 % 4) // 2).astype(jnp.float32)
    hl = l192 // NJ
    xi = x0 + dxv
    yi = y0 + dyv
    valid = ((xi >= 0.0) & (xi < float(W)) & (yi >= 0.0) & (yi < float(H)))
    xc = jnp.clip(xi, 0.0, float(W - 1)).astype(jnp.int32)
    yc = jnp.clip(yi, 0.0, float(H - 1)).astype(jnp.int32)
    idx_ref[0] = ((b * H + yc) * W + xc) * NH + hl
    wx = jnp.where(dxv == 0.0, 1.0 - fx, fx)
    wy = jnp.where(dyv == 0.0, 1.0 - fy, fy)
    wgt_ref[0] = awe * wx * wy * jnp.where(valid, 1.0, 0.0)


def _sc_gather_body(table_ref, idx_ref, g_ref, idx_v, rows_v, sem):
    wid = lax.axis_index("s") * 2 + lax.axis_index("c")
    base = wid * M_W

    def chunk(i, carry):
        cbase = base + i * CH
        pltpu.sync_copy(idx_ref.at[pl.ds(pl.multiple_of(cbase // 128, 8), KSUB)],
                        idx_v)
        copies = [
            pltpu.make_async_copy(table_ref.at[idx_v.at[k]],
                                  rows_v.at[pl.ds(k * 128, 128)], sem)
            for k in range(KSUB)
        ]
        for cp in copies:
            cp.start()
        for cp in copies:
            cp.wait()
        pltpu.sync_copy(rows_v, g_ref.at[pl.ds(cbase, CH)])
        return carry

    lax.fori_loop(0, NCHUNK, chunk, 0)


def _ln(x, g, b):
    m = jnp.mean(x, axis=-1, keepdims=True)
    xc = x - m
    v = jnp.mean(xc * xc, axis=-1, keepdims=True)
    return xc * lax.rsqrt(v + 1e-5) * g + b


def _reduce_ffn_body(g_ref, w_ref, q_ref, Wo_ref, bo_ref, W1_ref, b1_ref,
                     W2_ref, b2_ref, g1_ref, be1_ref, g2_ref, be2_ref, out_ref):
    q = q_ref[0]
    g = g_ref[0]
    w = w_ref[0]
    er = lax.broadcasted_iota(jnp.int32, (NJ, NJ * HD), 0)
    ec = lax.broadcasted_iota(jnp.int32, (NJ, NJ * HD), 1)
    E16 = (er == ec // HD).astype(jnp.float32)
    sr = lax.broadcasted_iota(jnp.int32, (NJ * HD, HD), 0)
    sc = lax.broadcasted_iota(jnp.int32, (NJ * HD, HD), 1)
    S512 = (sr % HD == sc).astype(jnp.float32)
    parts = []
    for h in range(NH):
        wh = w[:, h * NJ:(h + 1) * NJ]
        gh = g[:, h * NJ * HD:(h + 1) * NJ * HD]
        wE = jnp.dot(wh, E16, preferred_element_type=jnp.float32)
        parts.append(jnp.dot(wE * gh, S512,
                             preferred_element_type=jnp.float32))
    attn = jnp.concatenate(parts, axis=1)
    src2 = jnp.dot(attn, Wo_ref[...], preferred_element_type=jnp.float32) + bo_ref[...]
    h1 = _ln(q + src2, g1_ref[...], be1_ref[...])
    f = jnp.maximum(jnp.dot(h1, W1_ref[...], preferred_element_type=jnp.float32)
                    + b1_ref[...], 0.0)
    ff = jnp.dot(f, W2_ref[...], preferred_element_type=jnp.float32) + b2_ref[...]
    out_ref[0] = _ln(h1 + ff, g2_ref[...], be2_ref[...])


def kernel(src, Wso, bso, Waw, baw, Wv, bv, Wo, bo, W1, b1, W2, b2, g1, be1, g2, be2):
    q3 = src.reshape(B, N, C)

    full = lambda shape: pl.BlockSpec(shape, lambda *a: (0,) * len(shape))
    value, idx, wgt = pl.pallas_call(
        _prep_body,
        grid=(B,),
        in_specs=[
            pl.BlockSpec((1, N, C), lambda b: (b, 0, 0)),
            full((C, C)), full((1, C)),
            full((C, NH * NP * 2)), full((1, NH * NP * 2)),
            full((C, NH * NP)), full((1, NH * NP)),
        ],
        out_specs=[
            pl.BlockSpec((1, N, C), lambda b: (b, 0, 0)),
            pl.BlockSpec((1, N, NL), lambda b: (b, 0, 0)),
            pl.BlockSpec((1, N, NL), lambda b: (b, 0, 0)),
        ],
        out_shape=[
            jax.ShapeDtypeStruct((B, N, C), jnp.float32),
            jax.ShapeDtypeStruct((B, N, NL), jnp.int32),
            jax.ShapeDtypeStruct((B, N, NL), jnp.float32),
        ],
    )(q3, Wv, bv.reshape(1, C), Wso, bso.reshape(1, -1), Waw, baw.reshape(1, -1))

    table = value.reshape(B * N * NH, HD)
    idx2 = idx.reshape(M_TOT // 128, 128)

    sc_gather = pl.kernel(
        _sc_gather_body,
        out_type=jax.ShapeDtypeStruct((M_TOT, HD), jnp.float32),
        mesh=plsc.VectorSubcoreMesh(core_axis_name="c", subcore_axis_name="s",
                                    num_cores=2, num_subcores=16),
        scratch_types=[
            pltpu.VMEM((KSUB, 128), jnp.int32),
            pltpu.VMEM((CH, HD), jnp.float32),
            pltpu.SemaphoreType.DMA,
        ],
        compiler_params=pltpu.CompilerParams(use_tc_tiling_on_sc=False),
    )
    g = sc_gather(table, idx2)

    g3 = g.reshape(B, N, NL * HD)

    RB = 512
    out = pl.pallas_call(
        _reduce_ffn_body,
        grid=(B, N // RB),
        in_specs=[
            pl.BlockSpec((1, RB, NL * HD), lambda b, i: (b, i, 0)),
            pl.BlockSpec((1, RB, NL), lambda b, i: (b, i, 0)),
            pl.BlockSpec((1, RB, C), lambda b, i: (b, i, 0)),
            full((C, C)), full((1, C)),
            full((C, FF)), full((1, FF)),
            full((FF, C)), full((1, C)),
            full((1, C)), full((1, C)), full((1, C)), full((1, C)),
        ],
        out_specs=pl.BlockSpec((1, RB, C), lambda b, i: (b, i, 0)),
        out_shape=jax.ShapeDtypeStruct((B, N, C), jnp.float32),
    )(g3, wgt, q3, Wo, bo.reshape(1, C), W1, b1.reshape(1, FF), W2,
      b2.reshape(1, C), g1.reshape(1, C), be1.reshape(1, C), g2.reshape(1, C),
      be2.reshape(1, C))
    return out


# speedup vs baseline: 1.4174x; 1.0260x over previous
import jax
import jax.numpy as jnp
from jax import lax
from jax.experimental import pallas as pl
from jax.experimental.pallas import tpu as pltpu
from jax.experimental.pallas import tpu_sc as plsc

B, H, W, C = 4, 32, 32, 384
NH, NP = 12, 4
HD = C // NH
FF = 2048
N = H * W
NJ = NP * 4
NL = NH * NJ
M_TOT = B * N * NL

NWORK = 32
M_W = M_TOT // NWORK
CH = 3072
KSUB = CH // 128
NCHUNK = M_W // CH


def _prep_body(q_ref, Wv_ref, bv_ref, Wso_ref, bso_ref, Waw_ref, baw_ref,
               val_ref, idx_ref, wgt_ref):
    b = pl.program_id(0)
    q = q_ref[0]
    value = jnp.dot(q, Wv_ref[...], preferred_element_type=jnp.float32) + bv_ref[...]
    val_ref[0] = value
    off = jnp.dot(q, Wso_ref[...], preferred_element_type=jnp.float32) + bso_ref[...]
    awl = jnp.dot(q, Waw_ref[...], preferred_element_type=jnp.float32) + baw_ref[...]

    m = jnp.max(awl, axis=-1, keepdims=True)
    e = jnp.exp(awl - m)
    i48r = lax.broadcasted_iota(jnp.int32, (NH * NP, NH * NP), 0)
    i48c = lax.broadcasted_iota(jnp.int32, (NH * NP, NH * NP), 1)
    gsum = (i48r // NP == i48c // NP).astype(jnp.float32)
    denom = jnp.dot(e, gsum, preferred_element_type=jnp.float32)
    awn = e / denom

    n_row = lax.broadcasted_iota(jnp.int32, (N, 1), 0)
    colf = (n_row % W).astype(jnp.float32)
    rowf = (n_row // W).astype(jnp.float32)
    l96 = lax.broadcasted_iota(jnp.int32, (1, 2 * NH * NP), 1)
    is_x = (l96 % 2) == 0
    pix = off + jnp.where(is_x, colf, rowf)
    f0 = jnp.floor(pix)
    frac = pix - f0

    p96r = lax.broadcasted_iota(jnp.int32, (2 * NH * NP, NL), 0)
    p96c = lax.broadcasted_iota(jnp.int32, (2 * NH * NP, NL), 1)
    src = (p96c // NJ) * 8 + ((p96c % NJ) // 4) * 2
    Px = (p96r == src).astype(jnp.float32)
    Py = (p96r == src + 1).astype(jnp.float32)
    x0 = jnp.dot(f0, Px, preferred_element_type=jnp.float32)
    y0 = jnp.dot(f0, Py, preferred_element_type=jnp.float32)
    fx = jnp.dot(frac, Px, preferred_element_type=jnp.float32)
    fy = jnp.dot(frac, Py, preferred_element_type=jnp.float32)

    a48r = lax.broadcasted_iota(jnp.int32, (NH * NP, NL), 0)
    a48c = lax.broadcasted_iota(jnp.int32, (NH * NP, NL), 1)
    Paw = (a48r == a48c // 4).astype(jnp.float32)
    awe = jnp.dot(awn, Paw, preferred_element_type=jnp.float32)

    l192 = lax.broadcasted_iota(jnp.int32, (1, NL), 1)
    dxv = ((l192 % 4) % 2).astype(jnp.float32)
    dyv = ((l192 % 4) // 2).astype(jnp.float32)
    hl = l192 // NJ
    xi = x0 + dxv
    yi = y0 + dyv
    valid = ((xi >= 0.0) & (xi < float(W)) & (yi >= 0.0) & (yi < float(H)))
    xc = jnp.clip(xi, 0.0, float(W - 1)).astype(jnp.int32)
    yc = jnp.clip(yi, 0.0, float(H - 1)).astype(jnp.int32)
    idx_ref[0] = ((b * H + yc) * W + xc) * NH + hl
    wx = jnp.where(dxv == 0.0, 1.0 - fx, fx)
    wy = jnp.where(dyv == 0.0, 1.0 - fy, fy)
    wgt_ref[0] = awe * wx * wy * jnp.where(valid, 1.0, 0.0)


def _sc_gather_body(table_ref, idx_ref, g_ref, idx_v, rows_v, sem):
    wid = lax.axis_index("s") * 2 + lax.axis_index("c")
    base = wid * M_W

    def chunk(i, carry):
        cbase = base + i * CH
        pltpu.sync_copy(idx_ref.at[pl.ds(pl.multiple_of(cbase // 128, 8), KSUB)],
                        idx_v)
        copies = [
            pltpu.make_async_copy(table_ref.at[idx_v.at[k]],
                                  rows_v.at[pl.ds(k * 128, 128)], sem)
            for k in range(KSUB)
        ]
        for cp in copies:
            cp.start()
        for cp in copies:
            cp.wait()
        pltpu.sync_copy(rows_v, g_ref.at[pl.ds(cbase, CH)])
        return carry

    lax.fori_loop(0, NCHUNK, chunk, 0)


def _ln(x, g, b):
    m = jnp.mean(x, axis=-1, keepdims=True)
    xc = x - m
    v = jnp.mean(xc * xc, axis=-1, keepdims=True)
    return xc * lax.rsqrt(v + 1e-5) * g + b


def _reduce_ffn_body(g_ref, w_ref, q_ref, Wo_ref, bo_ref, W1_ref, b1_ref,
                     W2_ref, b2_ref, g1_ref, be1_ref, g2_ref, be2_ref, out_ref):
    q = q_ref[0]
    g = g_ref[0]
    w = w_ref[0]
    er = lax.broadcasted_iota(jnp.int32, (NJ, NJ * HD), 0)
    ec = lax.broadcasted_iota(jnp.int32, (NJ, NJ * HD), 1)
    E16 = (er == ec // HD).astype(jnp.float32)
    sr = lax.broadcasted_iota(jnp.int32, (NJ * HD, HD), 0)
    sc = lax.broadcasted_iota(jnp.int32, (NJ * HD, HD), 1)
    S512 = (sr % HD == sc).astype(jnp.float32)
    parts = []
    for h in range(NH):
        wh = w[:, h * NJ:(h + 1) * NJ]
        gh = g[:, h * NJ * HD:(h + 1) * NJ * HD]
        wE = jnp.dot(wh, E16, preferred_element_type=jnp.float32)
        parts.append(jnp.dot(wE * gh, S512,
                             preferred_element_type=jnp.float32))
    attn = jnp.concatenate(parts, axis=1)
    src2 = jnp.dot(attn, Wo_ref[...], preferred_element_type=jnp.float32) + bo_ref[...]
    h1 = _ln(q + src2, g1_ref[...], be1_ref[...])
    f = jnp.maximum(jnp.dot(h1, W1_ref[...], preferred_element_type=jnp.float32)
                    + b1_ref[...], 0.0)
    ff = jnp.dot(f, W2_ref[...], preferred_element_type=jnp.float32) + b2_ref[...]
    out_ref[0] = _ln(h1 + ff, g2_ref[...], be2_ref[...])


def kernel(src, Wso, bso, Waw, baw, Wv, bv, Wo, bo, W1, b1, W2, b2, g1, be1, g2, be2):
    q3 = src.reshape(B, N, C)

    full = lambda shape: pl.BlockSpec(shape, lambda *a: (0,) * len(shape))
    value, idx, wgt = pl.pallas_call(
        _prep_body,
        grid=(B,),
        in_specs=[
            pl.BlockSpec((1, N, C), lambda b: (b, 0, 0)),
            full((C, C)), full((1, C)),
            full((C, NH * NP * 2)), full((1, NH * NP * 2)),
            full((C, NH * NP)), full((1, NH * NP)),
        ],
        out_specs=[
            pl.BlockSpec((1, N, C), lambda b: (b, 0, 0)),
            pl.BlockSpec((1, N, NL), lambda b: (b, 0, 0)),
            pl.BlockSpec((1, N, NL), lambda b: (b, 0, 0)),
        ],
        out_shape=[
            jax.ShapeDtypeStruct((B, N, C), jnp.float32),
            jax.ShapeDtypeStruct((B, N, NL), jnp.int32),
            jax.ShapeDtypeStruct((B, N, NL), jnp.float32),
        ],
    )(q3, Wv, bv.reshape(1, C), Wso, bso.reshape(1, -1), Waw, baw.reshape(1, -1))

    table = value.reshape(B * N * NH, HD)
    idx2 = idx.reshape(M_TOT // 128, 128)

    sc_gather = pl.kernel(
        _sc_gather_body,
        out_type=jax.ShapeDtypeStruct((M_TOT, HD), jnp.float32),
        mesh=plsc.VectorSubcoreMesh(core_axis_name="c", subcore_axis_name="s",
                                    num_cores=2, num_subcores=16),
        scratch_types=[
            pltpu.VMEM((KSUB, 128), jnp.int32),
            pltpu.VMEM((CH, HD), jnp.float32),
            pltpu.SemaphoreType.DMA,
        ],
        compiler_params=pltpu.CompilerParams(use_tc_tiling_on_sc=False),
    )
    g = sc_gather(table, idx2)

    g3 = g.reshape(B, N, NL * HD)

    RB = 512
    out = pl.pallas_call(
        _reduce_ffn_body,
        grid=(B, N // RB),
        in_specs=[
            pl.BlockSpec((1, RB, NL * HD), lambda b, i: (b, i, 0)),
            pl.BlockSpec((1, RB, NL), lambda b, i: (b, i, 0)),
            pl.BlockSpec((1, RB, C), lambda b, i: (b, i, 0)),
            full((C, C)), full((1, C)),
            full((C, FF)), full((1, FF)),
            full((FF, C)), full((1, C)),
            full((1, C)), full((1, C)), full((1, C)), full((1, C)),
        ],
        out_specs=pl.BlockSpec((1, RB, C), lambda b, i: (b, i, 0)),
        out_shape=jax.ShapeDtypeStruct((B, N, C), jnp.float32),
    )(g3, wgt, q3, Wo, bo.reshape(1, C), W1, b1.reshape(1, FF), W2,
      b2.reshape(1, C), g1.reshape(1, C), be1.reshape(1, C), g2.reshape(1, C),
      be2.reshape(1, C))
    return out
